# baseline (speedup 1.0000x reference)
import functools

import jax
import jax.numpy as jnp
from jax import lax
from jax.experimental import pallas as pl
from jax.experimental.pallas import tpu as pltpu
from jax.experimental.pallas import tpu_sc as plsc

NC = 2
NS = 16
NW = NC * NS

B = 4096 * 200
BPW = B // NW
CH = 128
NCH = BPW // CH
KFLIGHT = 8
L = 16


def _body(xf_hbm, i0_hbm, i1_hbm, out_hbm, i0_v, i1_v, flat_v, out_v, sem):
    wid = lax.axis_index("s") * NC + lax.axis_index("c")
    base = wid * BPW

    pltpu.sync_copy(i0_hbm.at[pl.ds(base, BPW)], i0_v)
    pltpu.sync_copy(i1_hbm.at[pl.ds(base, BPW)], i1_v)

    def idx_body(i, carry):
        off = i * L
        a = i0_v[pl.ds(off, L)]
        b = i1_v[pl.ds(off, L)]
        flat_v[pl.ds(off, L)] = (a << 7) + b
        return carry

    lax.fori_loop(0, BPW // L, idx_body, 0, unroll=8)

    def gather_start(j):
        pltpu.async_copy(
            xf_hbm.at[flat_v.at[pl.ds(j * CH, CH)]],
            out_v.at[pl.ds(j * CH, CH)],
            sem,
        )

    def gather_wait(j):
        pltpu.make_async_copy(
            xf_hbm.at[flat_v.at[pl.ds(j * CH, CH)]],
            out_v.at[pl.ds(j * CH, CH)],
            sem,
        ).wait()

    for j in range(KFLIGHT):
        gather_start(j)

    def loop_body(j, carry):
        gather_wait(j)
        gather_start(j + KFLIGHT)
        return carry

    lax.fori_loop(0, NCH - KFLIGHT, loop_body, 0)

    def drain_body(j, carry):
        gather_wait(j)
        return carry

    lax.fori_loop(NCH - KFLIGHT, NCH, drain_body, 0)

    pltpu.sync_copy(out_v, out_hbm.at[pl.ds(base, BPW)])


@jax.jit
def kernel(x, idx0, idx1):
    out_shape = idx0.shape
    xf = x.reshape(-1)
    i0 = idx0.reshape(-1).astype(jnp.int32)
    i1 = idx1.reshape(-1).astype(jnp.int32)

    mesh = plsc.VectorSubcoreMesh(core_axis_name="c", subcore_axis_name="s")
    run = pl.kernel(
        _body,
        out_type=jax.ShapeDtypeStruct((B,), jnp.float32),
        mesh=mesh,
        scratch_types=[
            pltpu.VMEM((BPW,), jnp.int32),
            pltpu.VMEM((BPW,), jnp.int32),
            pltpu.VMEM((BPW,), jnp.int32),
            pltpu.VMEM((BPW,), jnp.float32),
            pltpu.SemaphoreType.DMA,
        ],
    )
    out = run(xf, i0, i1)
    return out.reshape(out_shape)


# speedup vs baseline: 1.1977x; 1.1977x over previous
import functools

import jax
import jax.numpy as jnp
from jax import lax
from jax.experimental import pallas as pl
from jax.experimental.pallas import tpu as pltpu
from jax.experimental.pallas import tpu_sc as plsc

NC = 2
NS = 16
NW = NC * NS

B = 4096 * 200
BPW = B // NW
CH = 512
NCH = BPW // CH
KFLIGHT = 8
L = 16


def _body(xf_hbm, i0_hbm, i1_hbm, out_hbm, i0_v, i1_v, flat_v, out_v, sem, isem):
    wid = lax.axis_index("s") * NC + lax.axis_index("c")
    base = wid * BPW

    cp0 = pltpu.async_copy(i0_hbm.at[pl.ds(base, BPW)], i0_v, isem)
    cp1 = pltpu.async_copy(i1_hbm.at[pl.ds(base, BPW)], i1_v, isem)
    cp0.wait()
    cp1.wait()

    def gather_start(j):
        pltpu.async_copy(
            xf_hbm.at[flat_v.at[pl.ds(j * CH, CH)]],
            out_v.at[pl.ds(j * CH, CH)],
            sem,
        )

    def gather_wait(j):
        pltpu.make_async_copy(
            xf_hbm.at[flat_v.at[pl.ds(j * CH, CH)]],
            out_v.at[pl.ds(j * CH, CH)],
            sem,
        ).wait()

    def loop_body(j, carry):
        off = j * CH

        def idx_body(i, c):
            o = off + i * L
            flat_v[pl.ds(o, L)] = (i0_v[pl.ds(o, L)] << 7) + i1_v[pl.ds(o, L)]
            return c

        lax.fori_loop(0, CH // L, idx_body, 0, unroll=8)

        @pl.when(j >= KFLIGHT)
        def _():
            gather_wait(j - KFLIGHT)

        gather_start(j)
        return carry

    lax.fori_loop(0, NCH, loop_body, 0)

    def drain_body(j, carry):
        gather_wait(j)
        return carry

    lax.fori_loop(NCH - KFLIGHT, NCH, drain_body, 0)

    pltpu.sync_copy(out_v, out_hbm.at[pl.ds(base, BPW)])


@jax.jit
def kernel(x, idx0, idx1):
    out_shape = idx0.shape
    xf = x.reshape(-1)
    i0 = idx0.reshape(-1).astype(jnp.int32)
    i1 = idx1.reshape(-1).astype(jnp.int32)

    mesh = plsc.VectorSubcoreMesh(core_axis_name="c", subcore_axis_name="s")
    run = pl.kernel(
        _body,
        out_type=jax.ShapeDtypeStruct((B,), jnp.float32),
        mesh=mesh,
        scratch_types=[
            pltpu.VMEM((BPW,), jnp.int32),
            pltpu.VMEM((BPW,), jnp.int32),
            pltpu.VMEM((BPW,), jnp.int32),
            pltpu.VMEM((BPW,), jnp.float32),
            pltpu.SemaphoreType.DMA,
            pltpu.SemaphoreType.DMA,
        ],
    )
    out = run(xf, i0, i1)
    return out.reshape(out_shape)


# speedup vs baseline: 1.2804x; 1.0691x over previous
import functools

import jax
import jax.numpy as jnp
from jax import lax
from jax.experimental import pallas as pl
from jax.experimental.pallas import tpu as pltpu
from jax.experimental.pallas import tpu_sc as plsc

NC = 2
NS = 16
NW = NC * NS

B = 4096 * 200
BPW = B // NW
CH = 1024
NCH = BPW // CH
KFLIGHT = 12
L = 16


def _body(xf_hbm, i0_hbm, i1_hbm, out_hbm, i0_v, i1_v, flat_v, out_v, sem, isem):
    wid = lax.axis_index("s") * NC + lax.axis_index("c")
    base = wid * BPW

    cp0 = pltpu.async_copy(i0_hbm.at[pl.ds(base, BPW)], i0_v, isem)
    cp1 = pltpu.async_copy(i1_hbm.at[pl.ds(base, BPW)], i1_v, isem)
    cp0.wait()
    cp1.wait()

    def gather_start(j):
        pltpu.async_copy(
            xf_hbm.at[flat_v.at[pl.ds(j * CH, CH)]],
            out_v.at[pl.ds(j * CH, CH)],
            sem,
        )

    def gather_wait(j):
        pltpu.make_async_copy(
            xf_hbm.at[flat_v.at[pl.ds(j * CH, CH)]],
            out_v.at[pl.ds(j * CH, CH)],
            sem,
        ).wait()

    def loop_body(j, carry):
        off = j * CH

        def idx_body(i, c):
            o = off + i * L
            flat_v[pl.ds(o, L)] = (i0_v[pl.ds(o, L)] << 7) + i1_v[pl.ds(o, L)]
            return c

        lax.fori_loop(0, CH // L, idx_body, 0, unroll=8)

        @pl.when(j >= KFLIGHT)
        def _():
            gather_wait(j - KFLIGHT)

        gather_start(j)
        return carry

    lax.fori_loop(0, NCH, loop_body, 0)

    def drain_body(j, carry):
        gather_wait(j)
        return carry

    lax.fori_loop(NCH - KFLIGHT, NCH, drain_body, 0)

    pltpu.sync_copy(out_v, out_hbm.at[pl.ds(base, BPW)])


@jax.jit
def kernel(x, idx0, idx1):
    out_shape = idx0.shape
    xf = x.reshape(-1)
    i0 = idx0.reshape(-1).astype(jnp.int32)
    i1 = idx1.reshape(-1).astype(jnp.int32)

    mesh = plsc.VectorSubcoreMesh(core_axis_name="c", subcore_axis_name="s")
    run = pl.kernel(
        _body,
        out_type=jax.ShapeDtypeStruct((B,), jnp.float32),
        mesh=mesh,
        scratch_types=[
            pltpu.VMEM((BPW,), jnp.int32),
            pltpu.VMEM((BPW,), jnp.int32),
            pltpu.VMEM((BPW,), jnp.int32),
            pltpu.VMEM((BPW,), jnp.float32),
            pltpu.SemaphoreType.DMA,
            pltpu.SemaphoreType.DMA,
        ],
    )
    out = run(xf, i0, i1)
    return out.reshape(out_shape)


# speedup vs baseline: 1.3724x; 1.0718x over previous
import jax
import jax.numpy as jnp
from jax import lax
from jax.experimental import pallas as pl
from jax.experimental.pallas import tpu as pltpu
from jax.experimental.pallas import tpu_sc as plsc

NC = 2
NS = 16
NW = NC * NS

NROW, NCOL = 4096, 200
PADCOL = 256
RPW = NROW // NW
BPW = RPW * NCOL
RPC = 8
CH = RPC * NCOL
NCH = BPW // CH
KFLIGHT = 8
L = 16
NSL = 13


def _body(xf_hbm, i0_hbm, i1_hbm, out_hbm, i0_v, i1_v, flat_v, out_v, sem, isem):
    wid = lax.axis_index("s") * NC + lax.axis_index("c")
    r0 = wid * RPW
    base = wid * BPW

    cp0 = pltpu.async_copy(i0_hbm.at[pl.ds(r0, RPW), :], i0_v, isem)
    cp1 = pltpu.async_copy(i1_hbm.at[pl.ds(r0, RPW), :], i1_v, isem)
    cp0.wait()
    cp1.wait()

    def gather_start(j):
        pltpu.async_copy(
            xf_hbm.at[flat_v.at[pl.ds(j * CH, CH)]],
            out_v.at[pl.ds(j * CH, CH)],
            sem,
        )

    def gather_wait(j):
        pltpu.make_async_copy(
            xf_hbm.at[flat_v.at[pl.ds(j * CH, CH)]],
            out_v.at[pl.ds(j * CH, CH)],
            sem,
        ).wait()

    def row_body(r, carry):
        fbase = r * NCOL
        for k in range(NSL):
            c = k * L
            flat_v[pl.ds(fbase + c, L)] = (
                (i0_v[r, pl.ds(c, L)] << 7) + i1_v[r, pl.ds(c, L)]
            )
        return carry

    def loop_body(j, carry):
        rlo = j * RPC
        lax.fori_loop(0, RPC, lambda i, c: row_body(rlo + i, c), carry)

        @pl.when(j >= KFLIGHT)
        def _():
            gather_wait(j - KFLIGHT)

        gather_start(j)
        return carry

    lax.fori_loop(0, NCH, loop_body, 0)

    def drain_body(j, carry):
        gather_wait(j)
        return carry

    lax.fori_loop(NCH - KFLIGHT, NCH, drain_body, 0)

    pltpu.sync_copy(out_v.at[pl.ds(0, BPW)], out_hbm.at[pl.ds(base, BPW)])


@jax.jit
def kernel(x, idx0, idx1):
    xf = x.reshape(-1)
    i0 = jnp.pad(idx0.astype(jnp.int32), ((0, 0), (0, PADCOL - NCOL)))
    i1 = jnp.pad(idx1.astype(jnp.int32), ((0, 0), (0, PADCOL - NCOL)))

    mesh = plsc.VectorSubcoreMesh(core_axis_name="c", subcore_axis_name="s")
    run = pl.kernel(
        _body,
        out_type=jax.ShapeDtypeStruct((NROW * NCOL,), jnp.float32),
        mesh=mesh,
        scratch_types=[
            pltpu.VMEM((RPW, PADCOL), jnp.int32),
            pltpu.VMEM((RPW, PADCOL), jnp.int32),
            pltpu.VMEM((BPW + L,), jnp.int32),
            pltpu.VMEM((BPW + L,), jnp.float32),
            pltpu.SemaphoreType.DMA,
            pltpu.SemaphoreType.DMA,
        ],
    )
    out = run(xf, i0, i1)
    return out.reshape(NROW, NCOL)
